# baseline (speedup 1.0000x reference)
import functools

import jax
import jax.numpy as jnp
from jax import lax
from jax.experimental import pallas as pl
from jax.experimental.pallas import tpu as pltpu
from jax.experimental.pallas import tpu_sc as plsc

V = 1000001
D = 64
NB = 16384
NT = 50
NC, NS = 2, 16
NW = NC * NS
BPW = NB // NW
NBB = BPW // 128
HB = 2
CH = HB * 128
NCHUNK = NT * (NBB // HB)


def _make_kernel():
    mesh = plsc.VectorSubcoreMesh(core_axis_name="c", subcore_axis_name="s")

    @functools.partial(
        pl.kernel,
        mesh=mesh,
        compiler_params=pltpu.CompilerParams(
            needs_layout_passes=False, use_tc_tiling_on_sc=False
        ),
        out_type=jax.ShapeDtypeStruct((NT, D // 8, NB // 128, 8, 128), jnp.float32),
        scratch_types=[
            pltpu.VMEM((NBB * NT, 128), jnp.int32),
            pltpu.VMEM((CH, D), jnp.float32),
            pltpu.VMEM((CH, D), jnp.float32),
            pltpu.VMEM((D // 8, HB, 8, 128), jnp.float32),
            pltpu.VMEM((D // 8, HB, 8, 128), jnp.float32),
            pltpu.SemaphoreType.DMA,
            pltpu.SemaphoreType.DMA,
            pltpu.SemaphoreType.DMA,
            pltpu.SemaphoreType.DMA,
        ],
    )
    def gather_kernel(
        table_hbm, idx_hbm, out_hbm,
        idx_v, rows0, rows1, cbuf0, cbuf1, gsem0, gsem1, osem0, osem1,
    ):
        wid = lax.axis_index("s") * NC + lax.axis_index("c")
        bb0 = wid * NBB
        rows = (rows0, rows1)
        cbuf = (cbuf0, cbuf1)
        gsem = (gsem0, gsem1)
        osem = (osem0, osem1)
        lane = lax.iota(jnp.int32, 16)

        pltpu.sync_copy(idx_hbm.at[pl.ds(bb0 * NT, NBB * NT)], idx_v)

        def fire_gather(c, p):
            t = c // 2
            h = c % 2
            for k in range(HB):
                pltpu.async_copy(
                    table_hbm.at[idx_v.at[(h * HB + k) * NT + t]],
                    rows[p].at[pl.ds(k * 128, 128)],
                    gsem[p],
                )

        def drain_gather(p):
            pltpu.make_async_copy(
                table_hbm.at[pl.ds(0, CH)], rows[p], gsem[p]
            ).wait()

        def fire_out(c, p):
            t = c // 2
            h = c % 2
            for cb in range(D // 8):
                pltpu.async_copy(
                    cbuf[p].at[cb],
                    out_hbm.at[t, cb, pl.ds(bb0 + h * HB, HB)],
                    osem[p],
                )

        def drain_out(p):
            for cb in range(D // 8):
                pltpu.make_async_copy(
                    cbuf[p].at[cb], out_hbm.at[0, cb, pl.ds(0, HB)], osem[p]
                ).wait()

        def transpose_chunk(c, p):
            t = c // 2
            h = c % 2

            def grp(i, carry):
                bbl = i // 8
                bmg = i % 8
                iv = plsc.load_gather(
                    idx_v,
                    [
                        jnp.full((16,), (h * HB + bbl) * NT + t, jnp.int32),
                        bmg * 16 + lane,
                    ],
                )
                m = jnp.where(iv == 0, jnp.float32(0.0), jnp.float32(1.0))
                rr = bbl * 128 + bmg * 16 + lane
                bbl_v = jnp.full((16,), bbl, jnp.int32)
                bm_v = bmg * 16 + lane
                for col in range(D):
                    x = plsc.load_gather(
                        rows[p], [rr, jnp.full((16,), col, jnp.int32)]
                    )
                    plsc.store_scatter(
                        cbuf[p],
                        [
                            jnp.full((16,), col // 8, jnp.int32),
                            bbl_v,
                            jnp.full((16,), col % 8, jnp.int32),
                            bm_v,
                        ],
                        x * m,
                    )
                return carry

            lax.fori_loop(0, HB * 8, grp, 0)

        fire_gather(0, 0)

        def step(c, p):
            pl.when(c + 1 < NCHUNK)(lambda: fire_gather(c + 1, 1 - p))
            pl.when(c >= 2)(lambda: drain_out(p))
            drain_gather(p)
            transpose_chunk(c, p)
            fire_out(c, p)

        def outer(u, carry):
            step(2 * u, 0)
            step(2 * u + 1, 1)
            return carry

        lax.fori_loop(0, NCHUNK // 2, outer, 0)
        drain_out(0)
        drain_out(1)

    return gather_kernel


@functools.lru_cache(maxsize=1)
def _get_kernel():
    return _make_kernel()


def kernel(ivectors, data):
    idx = (
        data.astype(jnp.int32)
        .T.reshape(NT, NB // 128, 128)
        .transpose(1, 0, 2)
        .reshape(NB // 128 * NT, 128)
    )
    out5 = _get_kernel()(ivectors, idx)
    return out5.transpose(2, 4, 0, 1, 3).reshape(NB, NT, D)


# speedup vs baseline: 1.2290x; 1.2290x over previous
import functools

import jax
import jax.numpy as jnp
from jax import lax
from jax.experimental import pallas as pl
from jax.experimental.pallas import tpu as pltpu
from jax.experimental.pallas import tpu_sc as plsc

V = 1000001
D = 64
NB = 16384
NT = 50
NC, NS = 2, 16
NW = NC * NS
BPW = NB // NW
NBB = BPW // 128
HB = 2
CH = HB * 128
NCHUNK = NT * (NBB // HB)
SLAB = HB * 8 * 128


def _make_kernel():
    mesh = plsc.VectorSubcoreMesh(core_axis_name="c", subcore_axis_name="s")

    @functools.partial(
        pl.kernel,
        mesh=mesh,
        compiler_params=pltpu.CompilerParams(
            needs_layout_passes=False, use_tc_tiling_on_sc=False
        ),
        out_type=jax.ShapeDtypeStruct((NT, D // 8, NB // 128, 8, 128), jnp.float32),
        scratch_types=[
            pltpu.VMEM((NBB * NT, 128), jnp.int32),
            pltpu.VMEM((CH, D), jnp.float32),
            pltpu.VMEM((CH, D), jnp.float32),
            pltpu.VMEM((D // 8, HB, 8, 128), jnp.float32),
            pltpu.VMEM((D // 8, HB, 8, 128), jnp.float32),
            pltpu.SemaphoreType.DMA,
            pltpu.SemaphoreType.DMA,
            pltpu.SemaphoreType.DMA,
            pltpu.SemaphoreType.DMA,
        ],
    )
    def gather_kernel(
        table_hbm, idx_hbm, out_hbm,
        idx_v, rows0, rows1, cbuf0, cbuf1, gsem0, gsem1, osem0, osem1,
    ):
        wid = lax.axis_index("s") * NC + lax.axis_index("c")
        bb0 = wid * NBB
        rows = (rows0, rows1)
        cbuf = (cbuf0, cbuf1)
        gsem = (gsem0, gsem1)
        osem = (osem0, osem1)
        lane = lax.iota(jnp.int32, 16)

        pltpu.sync_copy(idx_hbm.at[pl.ds(bb0 * NT, NBB * NT)], idx_v)

        def fire_gather(c, p):
            t = c // 2
            h = c % 2
            for k in range(HB):
                pltpu.async_copy(
                    table_hbm.at[idx_v.at[(h * HB + k) * NT + t]],
                    rows[p].at[pl.ds(k * 128, 128)],
                    gsem[p],
                )

        def drain_gather(p):
            pltpu.make_async_copy(
                table_hbm.at[pl.ds(0, CH)], rows[p], gsem[p]
            ).wait()

        def fire_out(c, p):
            t = c // 2
            h = c % 2
            for cb in range(D // 8):
                pltpu.async_copy(
                    cbuf[p].at[cb],
                    out_hbm.at[t, cb, pl.ds(bb0 + h * HB, HB)],
                    osem[p],
                )

        def drain_out(p):
            for cb in range(D // 8):
                pltpu.make_async_copy(
                    cbuf[p].at[cb], out_hbm.at[0, cb, pl.ds(0, HB)], osem[p]
                ).wait()

        kcol = [k * 16 + lane for k in range(D // 16)]
        pcb = [(k * 16 + lane) // 8 for k in range(D // 16)]
        pci = [(k * 16 + lane) % 8 for k in range(D // 16)]

        def detect_zero(c, p):
            t = c // 2
            h = c % 2
            mn = jnp.full((16,), 1, jnp.int32)
            for k in range(HB):
                for l in range(8):
                    mn = jnp.minimum(
                        mn,
                        plsc.load_gather(
                            idx_v,
                            [
                                jnp.full((16,), (h * HB + k) * NT + t, jnp.int32),
                                l * 16 + lane,
                            ],
                        ),
                    )
            nzero = plsc.all_reduce_population_count(mn == 0)
            return nzero[0] > 0

        def fix_zero_rows(c, p):
            t = c // 2
            h = c % 2

            def fixrow(r, carry):
                row = (h * HB + r // 128) * NT + t
                iv = plsc.load_gather(
                    idx_v,
                    [
                        jnp.full((16,), row, jnp.int32),
                        jnp.full((16,), r % 128, jnp.int32),
                    ],
                )
                m = jnp.where(iv == 0, jnp.float32(0.0), jnp.float32(1.0))
                rr = jnp.full((16,), r, jnp.int32)
                for k in range(D // 16):
                    x = plsc.load_gather(rows[p], [rr, kcol[k]])
                    plsc.store_scatter(rows[p], [rr, kcol[k]], x * m)
                return carry

            lax.fori_loop(0, CH, fixrow, 0)

        def transpose_chunk(p):
            def row_body(bm, carry):
                bm_v = jnp.full((16,), bm, jnp.int32)
                for bbl in range(HB):
                    r = bbl * 128 + bm
                    rsp = jnp.full((16,), r, jnp.int32)
                    bbl_v = jnp.full((16,), bbl, jnp.int32)
                    for k in range(D // 16):
                        x = plsc.load_gather(rows[p], [rsp, kcol[k]])
                        plsc.store_scatter(
                            cbuf[p], [pcb[k], bbl_v, pci[k], bm_v], x
                        )
                return carry

            lax.fori_loop(0, 128, row_body, 0)

        fire_gather(0, 0)

        def step(c, p):
            pl.when(c + 1 < NCHUNK)(lambda: fire_gather(c + 1, 1 - p))
            has_zero = detect_zero(c, p)
            pl.when(c >= 2)(lambda: drain_out(p))
            drain_gather(p)
            pl.when(has_zero)(lambda: fix_zero_rows(c, p))
            transpose_chunk(p)
            fire_out(c, p)

        def outer(u, carry):
            step(2 * u, 0)
            step(2 * u + 1, 1)
            return carry

        lax.fori_loop(0, NCHUNK // 2, outer, 0)
        drain_out(0)
        drain_out(1)

    return gather_kernel


@functools.lru_cache(maxsize=1)
def _get_kernel():
    return _make_kernel()


def kernel(ivectors, data):
    idx = (
        data.astype(jnp.int32)
        .T.reshape(NT, NB // 128, 128)
        .transpose(1, 0, 2)
        .reshape(NB // 128 * NT, 128)
    )
    out5 = _get_kernel()(ivectors, idx)
    return out5.transpose(2, 4, 0, 1, 3).reshape(NB, NT, D)


# speedup vs baseline: 1.4931x; 1.2149x over previous
import functools

import jax
import jax.numpy as jnp
from jax import lax
from jax.experimental import pallas as pl
from jax.experimental.pallas import tpu as pltpu
from jax.experimental.pallas import tpu_sc as plsc

V = 1000001
D = 64
NB = 16384
NT = 50
NC, NS = 2, 16
NW = NC * NS
BPW = NB // NW
NBB = BPW // 128
HB = 2
CH = HB * 128
NCHUNK = NT * (NBB // HB)
SLAB = HB * 8 * 128


def _make_kernel():
    mesh = plsc.VectorSubcoreMesh(core_axis_name="c", subcore_axis_name="s")

    @functools.partial(
        pl.kernel,
        mesh=mesh,
        compiler_params=pltpu.CompilerParams(
            needs_layout_passes=False, use_tc_tiling_on_sc=False
        ),
        out_type=jax.ShapeDtypeStruct((NT, D // 8, NB // 128, 8, 128), jnp.float32),
        scratch_types=[
            pltpu.VMEM((NBB * NT, 128), jnp.int32),
            pltpu.VMEM((CH, D), jnp.float32),
            pltpu.VMEM((CH, D), jnp.float32),
            pltpu.VMEM((D // 8, HB, 8, 128), jnp.float32),
            pltpu.VMEM((D // 8, HB, 8, 128), jnp.float32),
            pltpu.SemaphoreType.DMA,
            pltpu.SemaphoreType.DMA,
            pltpu.SemaphoreType.DMA,
            pltpu.SemaphoreType.DMA,
        ],
    )
    def gather_kernel(
        table_hbm, idx_hbm, out_hbm,
        idx_v, rows0, rows1, cbuf0, cbuf1, gsem0, gsem1, osem0, osem1,
    ):
        wid = lax.axis_index("s") * NC + lax.axis_index("c")
        bb0 = wid * NBB
        rows = (rows0, rows1)
        cbuf = (cbuf0, cbuf1)
        gsem = (gsem0, gsem1)
        osem = (osem0, osem1)
        lane = lax.iota(jnp.int32, 16)

        pltpu.sync_copy(idx_hbm.at[pl.ds(bb0 * NT, NBB * NT)], idx_v)

        def fire_gather(c, p):
            t = c // 2
            h = c % 2
            for k in range(HB):
                pltpu.async_copy(
                    table_hbm.at[idx_v.at[(h * HB + k) * NT + t]],
                    rows[p].at[pl.ds(k * 128, 128)],
                    gsem[p],
                )

        def drain_gather(p):
            pltpu.make_async_copy(
                table_hbm.at[pl.ds(0, CH)], rows[p], gsem[p]
            ).wait()

        def fire_out(c, p):
            t = c // 2
            h = c % 2
            for cb in range(D // 8):
                pltpu.async_copy(
                    cbuf[p].at[cb],
                    out_hbm.at[t, cb, pl.ds(bb0 + h * HB, HB)],
                    osem[p],
                )

        def drain_out(p):
            for cb in range(D // 8):
                pltpu.make_async_copy(
                    cbuf[p].at[cb], out_hbm.at[0, cb, pl.ds(0, HB)], osem[p]
                ).wait()

        kcol = [k * 16 + lane for k in range(D // 16)]
        pcb = [(k * 16 + lane) // 8 for k in range(D // 16)]
        pci = [(k * 16 + lane) % 8 for k in range(D // 16)]

        def detect_zero(c, p):
            t = c // 2
            h = c % 2
            mn = jnp.full((16,), 1, jnp.int32)
            for k in range(HB):
                for l in range(8):
                    mn = jnp.minimum(
                        mn,
                        plsc.load_gather(
                            idx_v,
                            [
                                jnp.full((16,), (h * HB + k) * NT + t, jnp.int32),
                                l * 16 + lane,
                            ],
                        ),
                    )
            nzero = plsc.all_reduce_population_count(mn == 0)
            return nzero[0] > 0

        def fix_zero_rows(c, p):
            t = c // 2
            h = c % 2

            def fixrow(r, carry):
                row = (h * HB + r // 128) * NT + t
                iv = plsc.load_gather(
                    idx_v,
                    [
                        jnp.full((16,), row, jnp.int32),
                        jnp.full((16,), r % 128, jnp.int32),
                    ],
                )
                m = jnp.where(iv == 0, jnp.float32(0.0), jnp.float32(1.0))
                rr = jnp.full((16,), r, jnp.int32)
                for k in range(D // 16):
                    x = plsc.load_gather(rows[p], [rr, kcol[k]])
                    plsc.store_scatter(rows[p], [rr, kcol[k]], x * m)
                return carry

            lax.fori_loop(0, CH, fixrow, 0)

        def transpose_chunk(p):
            @plsc.parallel_loop(0, 128, unroll=4)
            def row_body(bm):
                bm_v = jnp.full((16,), bm, jnp.int32)
                for bbl in range(HB):
                    r = bbl * 128 + bm
                    rsp = jnp.full((16,), r, jnp.int32)
                    bbl_v = jnp.full((16,), bbl, jnp.int32)
                    for k in range(D // 16):
                        x = plsc.load_gather(rows[p], [rsp, kcol[k]])
                        plsc.store_scatter(
                            cbuf[p], [pcb[k], bbl_v, pci[k], bm_v], x
                        )

        fire_gather(0, 0)

        def step(c, p):
            pl.when(c + 1 < NCHUNK)(lambda: fire_gather(c + 1, 1 - p))
            has_zero = detect_zero(c, p)
            pl.when(c >= 2)(lambda: drain_out(p))
            drain_gather(p)
            pl.when(has_zero)(lambda: fix_zero_rows(c, p))
            transpose_chunk(p)
            fire_out(c, p)

        def outer(u, carry):
            step(2 * u, 0)
            step(2 * u + 1, 1)
            return carry

        lax.fori_loop(0, NCHUNK // 2, outer, 0)
        drain_out(0)
        drain_out(1)

    return gather_kernel


@functools.lru_cache(maxsize=1)
def _get_kernel():
    return _make_kernel()


def kernel(ivectors, data):
    idx = (
        data.astype(jnp.int32)
        .T.reshape(NT, NB // 128, 128)
        .transpose(1, 0, 2)
        .reshape(NB // 128 * NT, 128)
    )
    out5 = _get_kernel()(ivectors, idx)
    return out5.transpose(2, 4, 0, 1, 3).reshape(NB, NT, D)
